# baseline (speedup 1.0000x reference)
import jax
import jax.numpy as jnp
from jax import lax
from jax.experimental import pallas as pl
from jax.experimental.pallas import tpu as pltpu

K = 1024
D = 256
BETA = 0.25
M_TILE = 1024
N_TOTAL = 16384


def _vq_kernel(zf_ref, e_ref, e2_ref,
               menc_ref, zq_ref, idx_ref, loss_ref, ppl_ref,
               counts_ref, loss_acc):
    i = pl.program_id(0)
    nsteps = pl.num_programs(0)
    zf = zf_ref[...]
    emb = e_ref[...]
    mm = lax.dot_general(zf, emb, (((1,), (1,)), ((), ())),
                         preferred_element_type=jnp.float32)
    zf2 = jnp.sum(zf * zf, axis=1, keepdims=True)
    d = zf2 + e2_ref[...] - 2.0 * mm
    mn = jnp.min(d, axis=1, keepdims=True)
    iota = lax.broadcasted_iota(jnp.int32, d.shape, 1).astype(jnp.float32)
    idxf = jnp.min(jnp.where(d == mn, iota, float(K)), axis=1, keepdims=True)
    one_hot = (iota == idxf).astype(jnp.float32)
    menc_ref[...] = one_hot
    idx = idxf[:, 0].astype(jnp.int32)
    zq = jnp.dot(one_hot, emb, preferred_element_type=jnp.float32)
    zq_ref[0] = jnp.transpose(zq)
    idx_ref[...] = idx.reshape(1, 1, M_TILE)

    part_loss = jnp.sum(mn)
    part_counts = jnp.dot(jnp.ones((1, M_TILE), jnp.float32), one_hot,
                          preferred_element_type=jnp.float32)

    @pl.when(i == 0)
    def _init():
        loss_acc[0, 0] = part_loss
        counts_ref[...] = part_counts

    @pl.when(i > 0)
    def _accum():
        loss_acc[0, 0] += part_loss
        counts_ref[...] += part_counts

    @pl.when(i == nsteps - 1)
    def _finish():
        loss_ref[...] = jnp.reshape(
            (1.0 + BETA) * loss_acc[0, 0] / (N_TOTAL * D), (1, 1))
        e_mean = counts_ref[...] * (1.0 / N_TOTAL)
        ppl_ref[...] = jnp.reshape(
            jnp.exp(-jnp.sum(e_mean * jnp.log(e_mean + 1e-10))), (1, 1))


def kernel(z, embedding):
    b, dz, h, w = z.shape
    zp = jnp.transpose(z, (0, 2, 3, 1))
    zf = zp.reshape(-1, D)
    e2 = jnp.sum(embedding ** 2, axis=1).reshape(1, K)
    n = zf.shape[0]
    nt = n // M_TILE
    out_shapes = (
        jax.ShapeDtypeStruct((n, K), jnp.float32),
        jax.ShapeDtypeStruct((b, D, h * w), jnp.float32),
        jax.ShapeDtypeStruct((nt, 1, M_TILE), jnp.int32),
        jax.ShapeDtypeStruct((1, 1), jnp.float32),
        jax.ShapeDtypeStruct((1, 1), jnp.float32),
    )
    menc, zq, idx, loss, ppl = pl.pallas_call(
        _vq_kernel,
        grid=(nt,),
        in_specs=[
            pl.BlockSpec((M_TILE, D), lambda i: (i, 0)),
            pl.BlockSpec((K, D), lambda i: (0, 0)),
            pl.BlockSpec((1, K), lambda i: (0, 0)),
        ],
        out_specs=[
            pl.BlockSpec((M_TILE, K), lambda i: (i, 0)),
            pl.BlockSpec((1, D, M_TILE), lambda i: (i, 0, 0)),
            pl.BlockSpec((1, 1, M_TILE), lambda i: (i, 0, 0)),
            pl.BlockSpec((1, 1), lambda i: (0, 0)),
            pl.BlockSpec((1, 1), lambda i: (0, 0)),
        ],
        out_shape=out_shapes,
        scratch_shapes=[pltpu.VMEM((1, K), jnp.float32),
                        pltpu.SMEM((1, 1), jnp.float32)],
    )(zf, embedding, e2)
    z_q_out = zq.reshape(b, D, h, w)
    return (loss[0, 0], z_q_out, ppl[0, 0], menc,
            idx.reshape(b, h, w))


# speedup vs baseline: 1.2955x; 1.2955x over previous
import jax
import jax.numpy as jnp
from jax import lax
from jax.experimental import pallas as pl
from jax.experimental.pallas import tpu as pltpu

K = 1024
D = 256
BETA = 0.25
M_TILE = 1024
N_TOTAL = 16384


def _vq_kernel(zf_ref, e_ref, e2_ref,
               menc_ref, zq_ref, idx_ref, loss_ref, ppl_ref,
               counts_ref, loss_acc):
    i = pl.program_id(0)
    nsteps = pl.num_programs(0)
    zf = zf_ref[...]
    emb = e_ref[...]
    zfx = zf + zf
    mm2 = lax.dot_general(zfx, emb, (((1,), (1,)), ((), ())),
                          preferred_element_type=jnp.float32)
    zf2 = jnp.sum(zf * zf, axis=1, keepdims=True)
    d = zf2 + e2_ref[...] - mm2
    mn = jnp.min(d, axis=1, keepdims=True)
    iota = lax.broadcasted_iota(jnp.int32, d.shape, 1).astype(jnp.float32)
    idxf = jnp.min(jnp.where(d == mn, iota, float(K)), axis=1, keepdims=True)
    one_hot = (iota == idxf).astype(jnp.float32)
    menc_ref[...] = one_hot
    idx = idxf[:, 0].astype(jnp.int32)
    zq = jnp.dot(one_hot, emb, preferred_element_type=jnp.float32)
    zq_ref[...] = zq
    idx_ref[...] = idx.reshape(1, 1, M_TILE)

    part_loss = jnp.sum(mn)
    part_counts = jnp.dot(jnp.ones((1, M_TILE), jnp.float32), one_hot,
                          preferred_element_type=jnp.float32)

    @pl.when(i == 0)
    def _init():
        loss_acc[0, 0] = part_loss
        counts_ref[...] = part_counts

    @pl.when(i > 0)
    def _accum():
        loss_acc[0, 0] += part_loss
        counts_ref[...] += part_counts

    @pl.when(i == nsteps - 1)
    def _finish():
        loss_ref[...] = jnp.reshape(
            (1.0 + BETA) * loss_acc[0, 0] / (N_TOTAL * D), (1, 1))
        e_mean = counts_ref[...] * (1.0 / N_TOTAL)
        ppl_ref[...] = jnp.reshape(
            jnp.exp(-jnp.sum(e_mean * jnp.log(e_mean + 1e-10))), (1, 1))


def kernel(z, embedding):
    b, dz, h, w = z.shape
    zp = jnp.transpose(z, (0, 2, 3, 1))
    zf = zp.reshape(-1, D)
    e2 = jnp.sum(embedding ** 2, axis=1).reshape(1, K)
    n = zf.shape[0]
    nt = n // M_TILE
    out_shapes = (
        jax.ShapeDtypeStruct((n, K), jnp.float32),
        jax.ShapeDtypeStruct((n, D), jnp.float32),
        jax.ShapeDtypeStruct((nt, 1, M_TILE), jnp.int32),
        jax.ShapeDtypeStruct((1, 1), jnp.float32),
        jax.ShapeDtypeStruct((1, 1), jnp.float32),
    )
    menc, zq, idx, loss, ppl = pl.pallas_call(
        _vq_kernel,
        grid=(nt,),
        in_specs=[
            pl.BlockSpec((M_TILE, D), lambda i: (i, 0)),
            pl.BlockSpec((K, D), lambda i: (0, 0)),
            pl.BlockSpec((1, K), lambda i: (0, 0)),
        ],
        out_specs=[
            pl.BlockSpec((M_TILE, K), lambda i: (i, 0)),
            pl.BlockSpec((M_TILE, D), lambda i: (i, 0)),
            pl.BlockSpec((1, 1, M_TILE), lambda i: (i, 0, 0)),
            pl.BlockSpec((1, 1), lambda i: (0, 0)),
            pl.BlockSpec((1, 1), lambda i: (0, 0)),
        ],
        out_shape=out_shapes,
        scratch_shapes=[pltpu.VMEM((1, K), jnp.float32),
                        pltpu.SMEM((1, 1), jnp.float32)],
    )(zf, embedding, e2)
    z_q_out = jnp.transpose(zq.reshape(b, h, w, D), (0, 3, 1, 2))
    return (loss[0, 0], z_q_out, ppl[0, 0], menc,
            idx.reshape(b, h, w))


# speedup vs baseline: 1.3218x; 1.0203x over previous
import jax
import jax.numpy as jnp
from jax import lax
from jax.experimental import pallas as pl
from jax.experimental.pallas import tpu as pltpu

K = 1024
D = 256
BETA = 0.25
M_TILE = 2048
N_TOTAL = 16384


def _vq_kernel(zf_ref, e_ref, e2_ref,
               menc_ref, zq_ref, idx_ref, loss_ref, ppl_ref,
               counts_ref, loss_acc):
    i = pl.program_id(0)
    nsteps = pl.num_programs(0)
    zf = zf_ref[...]
    emb = e_ref[...]
    zfx = zf + zf
    mm2 = lax.dot_general(zfx, emb, (((1,), (1,)), ((), ())),
                          preferred_element_type=jnp.float32)
    zf2 = jnp.sum(zf * zf, axis=1, keepdims=True)
    d = zf2 + e2_ref[...] - mm2
    mn = jnp.min(d, axis=1, keepdims=True)
    iota = lax.broadcasted_iota(jnp.int32, d.shape, 1).astype(jnp.float32)
    idxf = jnp.min(jnp.where(d == mn, iota, float(K)), axis=1, keepdims=True)
    one_hot = (iota == idxf).astype(jnp.float32)
    menc_ref[...] = one_hot
    idx = idxf[:, 0].astype(jnp.int32)
    zq = jnp.dot(one_hot, emb, preferred_element_type=jnp.float32)
    zq_ref[...] = zq
    idx_ref[...] = idx.reshape(1, 1, M_TILE)

    part_loss = jnp.sum(mn)
    part_counts = jnp.dot(jnp.ones((1, M_TILE), jnp.float32), one_hot,
                          preferred_element_type=jnp.float32)

    @pl.when(i == 0)
    def _init():
        loss_acc[0, 0] = part_loss
        counts_ref[...] = part_counts

    @pl.when(i > 0)
    def _accum():
        loss_acc[0, 0] += part_loss
        counts_ref[...] += part_counts

    @pl.when(i == nsteps - 1)
    def _finish():
        loss_ref[...] = jnp.reshape(
            (1.0 + BETA) * loss_acc[0, 0] / (N_TOTAL * D), (1, 1))
        e_mean = counts_ref[...] * (1.0 / N_TOTAL)
        ppl_ref[...] = jnp.reshape(
            jnp.exp(-jnp.sum(e_mean * jnp.log(e_mean + 1e-10))), (1, 1))


def kernel(z, embedding):
    b, dz, h, w = z.shape
    zp = jnp.transpose(z, (0, 2, 3, 1))
    zf = zp.reshape(-1, D)
    e2 = jnp.sum(embedding ** 2, axis=1).reshape(1, K)
    n = zf.shape[0]
    nt = n // M_TILE
    out_shapes = (
        jax.ShapeDtypeStruct((n, K), jnp.float32),
        jax.ShapeDtypeStruct((n, D), jnp.float32),
        jax.ShapeDtypeStruct((nt, 1, M_TILE), jnp.int32),
        jax.ShapeDtypeStruct((1, 1), jnp.float32),
        jax.ShapeDtypeStruct((1, 1), jnp.float32),
    )
    menc, zq, idx, loss, ppl = pl.pallas_call(
        _vq_kernel,
        grid=(nt,),
        in_specs=[
            pl.BlockSpec((M_TILE, D), lambda i: (i, 0)),
            pl.BlockSpec((K, D), lambda i: (0, 0)),
            pl.BlockSpec((1, K), lambda i: (0, 0)),
        ],
        out_specs=[
            pl.BlockSpec((M_TILE, K), lambda i: (i, 0)),
            pl.BlockSpec((M_TILE, D), lambda i: (i, 0)),
            pl.BlockSpec((1, 1, M_TILE), lambda i: (i, 0, 0)),
            pl.BlockSpec((1, 1), lambda i: (0, 0)),
            pl.BlockSpec((1, 1), lambda i: (0, 0)),
        ],
        out_shape=out_shapes,
        scratch_shapes=[pltpu.VMEM((1, K), jnp.float32),
                        pltpu.SMEM((1, 1), jnp.float32)],
    )(zf, embedding, e2)
    z_q_out = jnp.transpose(zq.reshape(b, h, w, D), (0, 3, 1, 2))
    return (loss[0, 0], z_q_out, ppl[0, 0], menc,
            idx.reshape(b, h, w))


# speedup vs baseline: 1.3612x; 1.0298x over previous
import jax
import jax.numpy as jnp
from jax import lax
from jax.experimental import pallas as pl
from jax.experimental.pallas import tpu as pltpu

K = 1024
D = 256
BETA = 0.25
M_TILE = 2048
N_TOTAL = 16384


def _vq_kernel(zf_ref, e_ref, e2_ref,
               menc_ref, zq_ref, idx_ref, loss_ref, ppl_ref,
               counts_ref, loss_acc):
    i = pl.program_id(0)
    nsteps = pl.num_programs(0)
    zf = zf_ref[...]
    emb = e_ref[...]
    zfx = zf + zf
    mm2 = lax.dot_general(zfx, emb, (((1,), (1,)), ((), ())),
                          preferred_element_type=jnp.float32)
    zf2 = jnp.sum(zf * zf, axis=1, keepdims=True)
    d = zf2 + e2_ref[...] - mm2
    idx = jnp.argmin(d, axis=1).astype(jnp.int32)
    iota = lax.broadcasted_iota(jnp.int32, d.shape, 1)
    one_hot = (iota == idx[:, None]).astype(jnp.float32)
    menc_ref[...] = one_hot
    zq = jnp.dot(one_hot, emb, preferred_element_type=jnp.float32)
    zq_ref[...] = zq
    idx_ref[...] = idx.reshape(1, 1, M_TILE)

    part_loss = jnp.sum((zq - zf) ** 2)
    part_counts = jnp.dot(jnp.ones((1, M_TILE), jnp.float32), one_hot,
                          preferred_element_type=jnp.float32)

    @pl.when(i == 0)
    def _init():
        loss_acc[0, 0] = part_loss
        counts_ref[...] = part_counts

    @pl.when(i > 0)
    def _accum():
        loss_acc[0, 0] += part_loss
        counts_ref[...] += part_counts

    @pl.when(i == nsteps - 1)
    def _finish():
        loss_ref[...] = jnp.reshape(
            (1.0 + BETA) * loss_acc[0, 0] / (N_TOTAL * D), (1, 1))
        e_mean = counts_ref[...] * (1.0 / N_TOTAL)
        ppl_ref[...] = jnp.reshape(
            jnp.exp(-jnp.sum(e_mean * jnp.log(e_mean + 1e-10))), (1, 1))


def kernel(z, embedding):
    b, dz, h, w = z.shape
    zp = jnp.transpose(z, (0, 2, 3, 1))
    zf = zp.reshape(-1, D)
    e2 = jnp.sum(embedding ** 2, axis=1).reshape(1, K)
    n = zf.shape[0]
    nt = n // M_TILE
    out_shapes = (
        jax.ShapeDtypeStruct((n, K), jnp.float32),
        jax.ShapeDtypeStruct((n, D), jnp.float32),
        jax.ShapeDtypeStruct((nt, 1, M_TILE), jnp.int32),
        jax.ShapeDtypeStruct((1, 1), jnp.float32),
        jax.ShapeDtypeStruct((1, 1), jnp.float32),
    )
    menc, zq, idx, loss, ppl = pl.pallas_call(
        _vq_kernel,
        grid=(nt,),
        in_specs=[
            pl.BlockSpec((M_TILE, D), lambda i: (i, 0)),
            pl.BlockSpec((K, D), lambda i: (0, 0)),
            pl.BlockSpec((1, K), lambda i: (0, 0)),
        ],
        out_specs=[
            pl.BlockSpec((M_TILE, K), lambda i: (i, 0)),
            pl.BlockSpec((M_TILE, D), lambda i: (i, 0)),
            pl.BlockSpec((1, 1, M_TILE), lambda i: (i, 0, 0)),
            pl.BlockSpec((1, 1), lambda i: (0, 0)),
            pl.BlockSpec((1, 1), lambda i: (0, 0)),
        ],
        out_shape=out_shapes,
        scratch_shapes=[pltpu.VMEM((1, K), jnp.float32),
                        pltpu.SMEM((1, 1), jnp.float32)],
    )(zf, embedding, e2)
    z_q_out = jnp.transpose(zq.reshape(b, h, w, D), (0, 3, 1, 2))
    return (loss[0, 0], z_q_out, ppl[0, 0], menc,
            idx.reshape(b, h, w))
